# baseline (speedup 1.0000x reference)
import functools

import jax
import jax.numpy as jnp
from jax import lax
from jax.experimental import pallas as pl
from jax.experimental.pallas import tpu as pltpu
from jax.experimental.pallas import tpu_sc as plsc

B = 16384
D = 32
NC, NS, L = 2, 16, 16
NW = NC * NS
BPW = B // NW
CHUNK = 128
NCH = BPW // CHUNK

MODULUS_WEIGHT = 4.0
PHASE_WEIGHT = 1.0

_INV_2PI = float(0.15915494309189535)
_MAGIC = float(12582912.0)
_PI_HI = float(3.1415927410125732)
_PI_LO = float(-8.742277657347586e-08)
_C3 = float(-1.0 / 6.0)
_C5 = float(1.0 / 120.0)
_C7 = float(-1.0 / 5040.0)
_C9 = float(1.0 / 362880.0)
_C11 = float(-1.0 / 39916800.0)


def _abs_sin_half(p):
    z = p * _INV_2PI
    k = (z + _MAGIC) - _MAGIC
    y = p * 0.5 - k * _PI_HI
    y = y - k * _PI_LO
    y2 = y * y
    t = _C9 + y2 * _C11
    t = _C7 + y2 * t
    t = _C5 + y2 * t
    t = _C3 + y2 * t
    t = 1.0 + y2 * t
    return jnp.abs(y * t)


def _newton_sqrt(x):
    i = lax.bitcast_convert_type(x, jnp.int32)
    i = jnp.int32(0x1FBD1DF5) + lax.shift_right_arithmetic(i, 1)
    y = lax.bitcast_convert_type(i, jnp.float32)
    for _ in range(3):
        y = 0.5 * (y + x / y)
    return y


def _sc_body(head, relation, tail, emt, ept, r_mod, r_ph, out, *scr):
    ih = scr[0:NCH]
    it = scr[NCH:2 * NCH]
    ir = scr[2 * NCH:3 * NCH]
    hm, hp, tm, tp = scr[3 * NCH:3 * NCH + 4]
    rm, rp = scr[3 * NCH + 4:3 * NCH + 6]
    out_v = scr[3 * NCH + 6]
    sem = scr[3 * NCH + 7]

    wid = lax.axis_index("s") * NC + lax.axis_index("c")
    base = wid * BPW

    for c in range(NCH):
        off = base + c * CHUNK
        pltpu.sync_copy(head.at[pl.ds(off, CHUNK)], ih[c])
        pltpu.sync_copy(tail.at[pl.ds(off, CHUNK)], it[c])
        pltpu.sync_copy(relation.at[pl.ds(off, CHUNK)], ir[c])

    for c in range(NCH):
        sl = pl.ds(c * CHUNK, CHUNK)
        pltpu.async_copy(r_mod.at[ir[c]], rm.at[sl], sem)
        pltpu.async_copy(r_ph.at[ir[c]], rp.at[sl], sem)

    def issue_body(f, carry):
        for c in range(NCH):
            dst = pl.ds(f * BPW + c * CHUNK, CHUNK)
            pltpu.async_copy(emt.at[f].at[ih[c]], hm.at[dst], sem)
            pltpu.async_copy(ept.at[f].at[ih[c]], hp.at[dst], sem)
            pltpu.async_copy(emt.at[f].at[it[c]], tm.at[dst], sem)
            pltpu.async_copy(ept.at[f].at[it[c]], tp.at[dst], sem)
        return carry

    lax.fori_loop(0, D, issue_body, 0)

    for buf in (hm, hp, tm, tp):
        pltpu.make_async_copy(out.at[pl.ds(0, B)], buf, sem).wait()
    for buf in (rm, rp):
        pltpu.make_async_copy(r_mod.at[pl.ds(0, BPW), :], buf, sem).wait()

    row_iota = lax.iota(jnp.int32, L)

    def group_body(g, carry):
        g16 = g * L
        rows = row_iota + g16
        acc_m = None
        acc_p = None
        for f in range(D):
            sl = pl.ds(f * BPW + g16, L)
            fcol = jnp.full((L,), f, jnp.int32)
            rmv = plsc.load_gather(rm, [rows, fcol])
            rpv = plsc.load_gather(rp, [rows, fcol])
            d = hm[sl] * rmv - tm[sl]
            sq = d * d
            acc_m = sq if acc_m is None else acc_m + sq
            s = _abs_sin_half(hp[sl] + rpv - tp[sl])
            acc_p = s if acc_p is None else acc_p + s
        score = MODULUS_WEIGHT * _newton_sqrt(acc_m) + PHASE_WEIGHT * acc_p
        out_v[pl.ds(g16, L)] = score
        return carry

    lax.fori_loop(0, BPW // L, group_body, 0)

    pltpu.sync_copy(out_v, out.at[pl.ds(base, BPW)])


@jax.jit
def _transee_score(head, relation, tail, emt, ept, r_mod, r_ph):
    mesh = plsc.VectorSubcoreMesh(core_axis_name="c", subcore_axis_name="s")
    scratch = (
        [pltpu.VMEM((CHUNK,), jnp.int32)] * (3 * NCH)
        + [pltpu.VMEM((B,), jnp.float32)] * 4
        + [pltpu.VMEM((BPW, D), jnp.float32)] * 2
        + [pltpu.VMEM((BPW,), jnp.float32)]
        + [pltpu.SemaphoreType.DMA]
    )
    return pl.kernel(
        _sc_body,
        out_type=jax.ShapeDtypeStruct((B,), jnp.float32),
        mesh=mesh,
        scratch_types=scratch,
        compiler_params=pltpu.CompilerParams(needs_layout_passes=False,
                                             use_tc_tiling_on_sc=False),
    )(head, relation, tail, emt, ept, r_mod, r_ph)


def kernel(head, relation, tail, entity_modulus, entity_phase,
           relation_modulus, relation_phase):
    return _transee_score(head, relation, tail,
                          entity_modulus.T, entity_phase.T,
                          relation_modulus, relation_phase)
